# baseline (speedup 1.0000x reference)
import functools

import numpy as np
import jax
import jax.numpy as jnp
from jax import lax
from jax.experimental import pallas as pl
from jax.experimental.pallas import tpu as pltpu
from jax.experimental.pallas import tpu_sc as plsc

_B = 16384
_NUM_DENSE = 13
_NF = 26
_VOCAB = 100000
_D = 64
_NI = _NF + 1
_BOTTOM_IN = 16
_TOP_IN = 416

_ROWS_PER_DMA = 128
_NW = 32


def _sc_gather(table, idx2d):
    rows_total = idx2d.shape[0] * idx2d.shape[1]
    per_w = rows_total // _NW
    chunks = per_w // _ROWS_PER_DMA
    mesh = plsc.VectorSubcoreMesh(core_axis_name="c", subcore_axis_name="s")

    @functools.partial(
        pl.kernel,
        mesh=mesh,
        out_type=jax.ShapeDtypeStruct((rows_total, _D), jnp.float32),
        scratch_types=[
            pltpu.VMEM((chunks, _ROWS_PER_DMA), jnp.int32),
            pltpu.VMEM((_ROWS_PER_DMA, _D), jnp.float32),
            pltpu.SemaphoreType.DMA,
        ],
        compiler_params=pltpu.CompilerParams(use_tc_tiling_on_sc=False),
    )
    def gather_k(table_hbm, idx_hbm, out_hbm, idx_v, rows_v, sem):
        wid = lax.axis_index("s") * 2 + lax.axis_index("c")
        pltpu.sync_copy(idx_hbm.at[pl.ds(wid * chunks, chunks)], idx_v)

        def body(g, carry):
            pltpu.async_copy(table_hbm.at[idx_v.at[g]], rows_v, sem).wait()
            pltpu.sync_copy(
                rows_v,
                out_hbm.at[pl.ds((wid * chunks + g) * _ROWS_PER_DMA,
                                 _ROWS_PER_DMA)],
            )
            return carry

        lax.fori_loop(0, chunks, body, 0)

    return gather_k(table, idx2d)


def _dense_body(x_ref, emb_ref,
                bw0_ref, bb0_ref, bw1_ref, bb1_ref, bw2_ref, bb2_ref,
                w0b_ref, w0z_ref, tb0_ref,
                tw1_ref, tb1_ref, tw2_ref, tb2_ref,
                tw3_ref, tb3_ref, tw4_ref, tb4_ref,
                out_ref):
    f32 = jnp.float32
    x = x_ref[...]
    h = jnp.maximum(jnp.dot(x, bw0_ref[...], preferred_element_type=f32)
                    + bb0_ref[...], 0.0)
    h = jnp.maximum(jnp.dot(h, bw1_ref[...], preferred_element_type=f32)
                    + bb1_ref[...], 0.0)
    bot = jnp.maximum(jnp.dot(h, bw2_ref[...], preferred_element_type=f32)
                      + bb2_ref[...], 0.0)

    emb = emb_ref[...]
    x3 = jnp.concatenate([bot[:, None, :], emb], axis=1)
    z = lax.dot_general(x3, x3, (((2,), (2,)), ((0,), (0,))),
                        preferred_element_type=f32)
    zf = z.reshape(z.shape[0], _NI * _NI)

    h1 = (jnp.dot(bot, w0b_ref[...], preferred_element_type=f32)
          + jnp.dot(zf, w0z_ref[...], preferred_element_type=f32)
          + tb0_ref[...])
    h1 = jnp.maximum(h1, 0.0)
    h2 = jnp.maximum(jnp.dot(h1, tw1_ref[...], preferred_element_type=f32)
                     + tb1_ref[...], 0.0)
    h3 = jnp.maximum(jnp.dot(h2, tw2_ref[...], preferred_element_type=f32)
                     + tb2_ref[...], 0.0)
    h4 = jnp.maximum(jnp.dot(h3, tw3_ref[...], preferred_element_type=f32)
                     + tb3_ref[...], 0.0)
    out_ref[...] = (jnp.dot(h4, tw4_ref[...], preferred_element_type=f32)
                    + tb4_ref[...])


def _dense(x0, emb3, bw0, bb0, bw1, bb1, bw2, bb2,
           w0b, w0z, tb0, tw1, tb1, tw2, tb2, tw3, tb3, tw4, tb4):
    bt = 512
    grid = (_B // bt,)

    full = lambda b: (0, 0)
    full1 = lambda b: (0,)

    return pl.pallas_call(
        _dense_body,
        grid=grid,
        in_specs=[
            pl.BlockSpec((bt, _BOTTOM_IN), lambda b: (b, 0)),
            pl.BlockSpec((bt, _NF, _D), lambda b: (b, 0, 0)),
            pl.BlockSpec(bw0.shape, full), pl.BlockSpec(bb0.shape, full1),
            pl.BlockSpec(bw1.shape, full), pl.BlockSpec(bb1.shape, full1),
            pl.BlockSpec(bw2.shape, full), pl.BlockSpec(bb2.shape, full1),
            pl.BlockSpec(w0b.shape, full), pl.BlockSpec(w0z.shape, full),
            pl.BlockSpec(tb0.shape, full1),
            pl.BlockSpec(tw1.shape, full), pl.BlockSpec(tb1.shape, full1),
            pl.BlockSpec(tw2.shape, full), pl.BlockSpec(tb2.shape, full1),
            pl.BlockSpec(tw3.shape, full), pl.BlockSpec(tb3.shape, full1),
            pl.BlockSpec(tw4.shape, full), pl.BlockSpec(tb4.shape, full1),
        ],
        out_specs=pl.BlockSpec((bt, 1), lambda b: (b, 0)),
        out_shape=jax.ShapeDtypeStruct((_B, 1), jnp.float32),
        compiler_params=pltpu.CompilerParams(
            dimension_semantics=("arbitrary",),
        ),
    )(x0, emb3, bw0, bb0, bw1, bb1, bw2, bb2,
      w0b, w0z, tb0, tw1, tb1, tw2, tb2, tw3, tb3, tw4, tb4)


def kernel(numerical_features, categorical_features, embedding_tables,
           bw0, bb0, bw1, bb1, bw2, bb2,
           tw0, tb0, tw1, tb1, tw2, tb2, tw3, tb3, tw4, tb4):
    b = numerical_features.shape[0]

    table = embedding_tables.reshape(_NF * _VOCAB, _D)
    offsets = (jnp.arange(_NF, dtype=jnp.int32) * _VOCAB)[None, :]
    idx2d = (categorical_features + offsets).reshape(-1, _ROWS_PER_DMA)

    x0 = jnp.concatenate(
        [numerical_features,
         jnp.zeros((b, _BOTTOM_IN - _NUM_DENSE), jnp.float32)], axis=1)

    li, lj = np.tril_indices(_NI, -1)
    w0b = tw0[:_D]
    w0z = jnp.zeros((_NI * _NI, tw0.shape[1]), jnp.float32)
    w0z = w0z.at[li * _NI + lj].set(tw0[_D:_D + li.shape[0]])

    rows = _sc_gather(table, idx2d)
    emb3 = rows.reshape(b, _NF, _D)

    return _dense(x0, emb3, bw0, bb0, bw1, bb1, bw2, bb2,
                  w0b, w0z, tb0, tw1, tb1, tw2, tb2, tw3, tb3, tw4, tb4)
